# baseline (speedup 1.0000x reference)
import jax
import jax.numpy as jnp
from jax import lax
from jax.experimental import pallas as pl
from jax.experimental.pallas import tpu as pltpu
from jax.experimental.pallas import tpu_sc as plsc

_N_FIELDS = 26
_VOCAB = 100000
_N_ROWS = _VOCAB + 1
_EMBED = 16
_BATCH = 16384

_NW = 32
_NU = _N_FIELDS * _EMBED
_UPT = _NU // _NW
_BC = 4096
_L = 16


def _idx_body(z_ref, i_ref):
    f = pl.program_id(0)
    z = z_ref[pl.ds(f, 1)][0]
    nan = jnp.isnan(z)
    i_ref[...] = jnp.where(nan, jnp.int32(_VOCAB),
                           jnp.where(nan, jnp.float32(0), z).astype(jnp.int32))


_NCH = _BATCH // _BC


def _lookup_body(idx_hbm, tab_hbm, out_hbm, plane_v, idxf_v, obuf_v,
                 psem, isem, osem):
    wid = lax.axis_index("s") * 2 + lax.axis_index("c")

    def ufe(k):
        u = wid * _UPT + k
        return u, u // _EMBED, u % _EMBED

    _, f0, e0 = ufe(0)
    plane_cp = pltpu.async_copy(tab_hbm.at[f0, e0], plane_v, psem)
    pltpu.sync_copy(idx_hbm.at[pl.ds(f0 * _BATCH, _BATCH)], idxf_v)
    out_cps = []

    for k in range(_UPT):
        u, f, e = ufe(k)
        if k > 0:
            @pl.when(e == 0)
            def _():
                pltpu.sync_copy(idx_hbm.at[pl.ds(f * _BATCH, _BATCH)], idxf_v)
        plane_cp.wait()
        for c in range(_NCH):
            buf = c % 2
            if len(out_cps) >= 2:
                out_cps[-2].wait()
            base = c * _BC

            def gat(j, _):
                v = idxf_v[pl.ds(base + j * _L, _L)]
                obuf_v[buf, pl.ds(j * _L, _L)] = plsc.load_gather(plane_v, [v])
                return ()

            lax.fori_loop(0, _BC // _L, gat, (), unroll=16)

            if c == _NCH - 1 and k + 1 < _UPT:
                _, f2, e2 = ufe(k + 1)
                plane_cp = pltpu.async_copy(tab_hbm.at[f2, e2], plane_v, psem)
            out_cps.append(pltpu.async_copy(
                obuf_v.at[buf], out_hbm.at[u, pl.ds(base, _BC)], osem))

    for cp in out_cps[-2:]:
        cp.wait()


@jax.jit
def _embed_lookup(Z_vec, tables):
    tab_t = jnp.transpose(tables, (0, 2, 1))
    z_t = jnp.transpose(Z_vec)

    idx1d = pl.pallas_call(
        _idx_body,
        grid=(_N_FIELDS,),
        in_specs=[pl.BlockSpec((_N_FIELDS, _BATCH), lambda f: (0, 0))],
        out_specs=pl.BlockSpec((_BATCH,), lambda f: (f,)),
        out_shape=jax.ShapeDtypeStruct((_N_FIELDS * _BATCH,), jnp.int32),
    )(z_t)

    mesh = plsc.VectorSubcoreMesh(core_axis_name="c", subcore_axis_name="s")
    kfn = pl.kernel(
        _lookup_body,
        out_type=jax.ShapeDtypeStruct((_NU, _BATCH), jnp.float32),
        mesh=mesh,
        scratch_types=[
            pltpu.VMEM((_N_ROWS,), jnp.float32),
            pltpu.VMEM((_BATCH,), jnp.int32),
            pltpu.VMEM((2, _BC), jnp.float32),
            pltpu.SemaphoreType.DMA,
            pltpu.SemaphoreType.DMA,
            pltpu.SemaphoreType.DMA,
        ],
        compiler_params=pltpu.CompilerParams(
            use_tc_tiling_on_sc=True, needs_layout_passes=False),
    )
    out_t = kfn(idx1d, tab_t)
    return jnp.transpose(out_t)


def kernel(Z_vec, tables):
    return _embed_lookup(Z_vec, tables)


# speedup vs baseline: 1.0029x; 1.0029x over previous
import jax
import jax.numpy as jnp
from jax import lax
from jax.experimental import pallas as pl
from jax.experimental.pallas import tpu as pltpu
from jax.experimental.pallas import tpu_sc as plsc

_N_FIELDS = 26
_VOCAB = 100000
_N_ROWS = _VOCAB + 1
_EMBED = 16
_BATCH = 16384

_NW = 32
_NU = _N_FIELDS * _EMBED
_UPT = _NU // _NW
_BC = 4096
_L = 16


def _idx_body(z_ref, i_ref):
    f = pl.program_id(0)
    z = z_ref[pl.ds(f, 1)][0]
    nan = jnp.isnan(z)
    i_ref[...] = jnp.where(nan, jnp.int32(_VOCAB),
                           jnp.where(nan, jnp.float32(0), z).astype(jnp.int32))


_NCH = _BATCH // _BC


def _lookup_body(idx_hbm, tab_hbm, out_hbm, plane_v, idxf_v, obuf_v,
                 psem, isem, osem):
    wid = lax.axis_index("s") * 2 + lax.axis_index("c")

    def ufe(k):
        u = wid * _UPT + k
        return u, u // _EMBED, u % _EMBED

    _, f0, e0 = ufe(0)
    plane_cp = pltpu.async_copy(tab_hbm.at[f0, e0], plane_v, psem)
    pltpu.sync_copy(idx_hbm.at[pl.ds(f0 * _BATCH, _BATCH)], idxf_v)
    out_cps = []

    for k in range(_UPT):
        u, f, e = ufe(k)
        if k > 0:
            @pl.when(e == 0)
            def _():
                pltpu.sync_copy(idx_hbm.at[pl.ds(f * _BATCH, _BATCH)], idxf_v)
        plane_cp.wait()
        for c in range(_NCH):
            buf = c % 2
            if len(out_cps) >= 2:
                out_cps[-2].wait()
            base = c * _BC

            def gat(j, _):
                v = idxf_v[pl.ds(base + j * _L, _L)]
                obuf_v[buf, pl.ds(j * _L, _L)] = plsc.load_gather(plane_v, [v])
                return ()

            lax.fori_loop(0, _BC // _L, gat, (), unroll=8)

            if c == _NCH - 1 and k + 1 < _UPT:
                _, f2, e2 = ufe(k + 1)
                plane_cp = pltpu.async_copy(tab_hbm.at[f2, e2], plane_v, psem)
            out_cps.append(pltpu.async_copy(
                obuf_v.at[buf], out_hbm.at[u, pl.ds(base, _BC)], osem))

    for cp in out_cps[-2:]:
        cp.wait()


@jax.jit
def _embed_lookup(Z_vec, tables):
    tab_t = jnp.transpose(tables, (0, 2, 1))
    z_t = jnp.transpose(Z_vec)

    idx1d = pl.pallas_call(
        _idx_body,
        grid=(_N_FIELDS,),
        in_specs=[pl.BlockSpec((_N_FIELDS, _BATCH), lambda f: (0, 0))],
        out_specs=pl.BlockSpec((_BATCH,), lambda f: (f,)),
        out_shape=jax.ShapeDtypeStruct((_N_FIELDS * _BATCH,), jnp.int32),
    )(z_t)

    mesh = plsc.VectorSubcoreMesh(core_axis_name="c", subcore_axis_name="s")
    kfn = pl.kernel(
        _lookup_body,
        out_type=jax.ShapeDtypeStruct((_NU, _BATCH), jnp.float32),
        mesh=mesh,
        scratch_types=[
            pltpu.VMEM((_N_ROWS,), jnp.float32),
            pltpu.VMEM((_BATCH,), jnp.int32),
            pltpu.VMEM((2, _BC), jnp.float32),
            pltpu.SemaphoreType.DMA,
            pltpu.SemaphoreType.DMA,
            pltpu.SemaphoreType.DMA,
        ],
        compiler_params=pltpu.CompilerParams(
            use_tc_tiling_on_sc=True, needs_layout_passes=False),
    )
    out_t = kfn(idx1d, tab_t)
    return jnp.transpose(out_t)


def kernel(Z_vec, tables):
    return _embed_lookup(Z_vec, tables)
